# baseline (speedup 1.0000x reference)
import jax
import jax.numpy as jnp
from jax.experimental import pallas as pl
from jax.experimental.pallas import tpu as pltpu

_BM = 400


def _proj(feat_ref, w_ref, y_ref):
    y_ref[...] = jnp.dot(
        feat_ref[...], w_ref[...], preferred_element_type=jnp.float32
    )


def _spmm(adj_ref, y_ref, out_ref):
    out_ref[...] = jnp.dot(
        adj_ref[...], y_ref[...], preferred_element_type=jnp.float32
    )


def kernel(feat, adj, weight):
    n, in_feat = feat.shape
    out_feat = weight.shape[1]
    y = pl.pallas_call(
        _proj,
        grid=(1,),
        in_specs=[
            pl.BlockSpec((n, in_feat), lambda i: (0, 0)),
            pl.BlockSpec((in_feat, out_feat), lambda i: (0, 0)),
        ],
        out_specs=pl.BlockSpec((n, out_feat), lambda i: (0, 0)),
        out_shape=jax.ShapeDtypeStruct((n, out_feat), jnp.float32),
    )(feat, weight)
    bm = _BM if n % _BM == 0 else n
    return pl.pallas_call(
        _spmm,
        grid=(n // bm,),
        in_specs=[
            pl.BlockSpec((bm, n), lambda i: (i, 0)),
            pl.BlockSpec((n, out_feat), lambda i: (0, 0)),
        ],
        out_specs=pl.BlockSpec((bm, out_feat), lambda i: (i, 0)),
        out_shape=jax.ShapeDtypeStruct((n, out_feat), jnp.float32),
        compiler_params=pltpu.CompilerParams(
            dimension_semantics=("parallel",)
        ),
    )(adj, y)


# speedup vs baseline: 1.0441x; 1.0441x over previous
import jax
import jax.numpy as jnp
from jax.experimental import pallas as pl
from jax.experimental.pallas import tpu as pltpu

_BM = 200


def _fused(feat_ref, w_ref, a0_ref, a1_ref, out_ref, y_ref):
    @pl.when(pl.program_id(0) == 0)
    def _():
        y_ref[...] = jnp.dot(
            feat_ref[...], w_ref[...], preferred_element_type=jnp.float32
        )

    out_ref[0] = jnp.dot(a0_ref[...], y_ref[...], preferred_element_type=jnp.float32)
    out_ref[1] = jnp.dot(a1_ref[...], y_ref[...], preferred_element_type=jnp.float32)


def kernel(feat, adj, weight):
    n, in_feat = feat.shape
    out_feat = weight.shape[1]
    half = n // 2
    bm = _BM if half % _BM == 0 else half
    steps = half // bm
    out = pl.pallas_call(
        _fused,
        grid=(steps,),
        in_specs=[
            pl.BlockSpec((n, in_feat), lambda i: (0, 0)),
            pl.BlockSpec((in_feat, out_feat), lambda i: (0, 0)),
            pl.BlockSpec((bm, n), lambda i: (i, 0)),
            pl.BlockSpec((bm, n), lambda i, s=steps: (i + s, 0)),
        ],
        out_specs=pl.BlockSpec((2, bm, out_feat), lambda i: (0, i, 0)),
        out_shape=jax.ShapeDtypeStruct((2, half, out_feat), jnp.float32),
        scratch_shapes=[pltpu.VMEM((n, out_feat), jnp.float32)],
    )(feat, weight, adj, adj)
    return out.reshape(n, out_feat)


# speedup vs baseline: 1.0559x; 1.0113x over previous
import jax
import jax.numpy as jnp
from jax.experimental import pallas as pl
from jax.experimental.pallas import tpu as pltpu

_BM = 400


def _fused(feat_ref, w_ref, adj_ref, out_ref, y_ref):
    @pl.when(pl.program_id(0) == 0)
    def _():
        y_ref[...] = jnp.dot(
            feat_ref[...], w_ref[...], preferred_element_type=jnp.float32
        ).astype(jnp.bfloat16)

    out_ref[...] = jnp.dot(
        adj_ref[...].astype(jnp.bfloat16),
        y_ref[...],
        preferred_element_type=jnp.float32,
    )


def kernel(feat, adj, weight):
    n, in_feat = feat.shape
    out_feat = weight.shape[1]
    bm = _BM if n % _BM == 0 else n
    return pl.pallas_call(
        _fused,
        grid=(n // bm,),
        in_specs=[
            pl.BlockSpec((n, in_feat), lambda i: (0, 0)),
            pl.BlockSpec((in_feat, out_feat), lambda i: (0, 0)),
            pl.BlockSpec((bm, n), lambda i: (i, 0)),
        ],
        out_specs=pl.BlockSpec((bm, out_feat), lambda i: (i, 0)),
        out_shape=jax.ShapeDtypeStruct((n, out_feat), jnp.float32),
        scratch_shapes=[pltpu.VMEM((n, out_feat), jnp.bfloat16)],
    )(feat, weight, adj)
